# baseline (speedup 1.0000x reference)
import functools

import jax
import jax.numpy as jnp
from jax import lax
from jax.experimental import pallas as pl
from jax.experimental.pallas import tpu as pltpu
from jax.experimental.pallas import tpu_sc as plsc

D = 1024
NTOK = 8192
NWORK = 32
TPW = NTOK // NWORK
C = 8
NCHUNK = TPW // C
NV = D // 16
EPS = 1e-12


def _rsqrt16(v):
    i = lax.bitcast_convert_type(v, jnp.int32)
    i = jnp.int32(0x5F3759DF) - (i >> 1)
    y = lax.bitcast_convert_type(i, jnp.float32)
    for _ in range(3):
        y = y * (1.5 - 0.5 * v * y * y)
    return y


def _sc_body(idx_hbm, table_hbm, gamma_hbm, beta_hbm, out_hbm,
             idx_v, rows_v, buf_v, gam_v, bet_v, sem):
    wid = lax.axis_index("s") * 2 + lax.axis_index("c")
    tok0 = wid * TPW

    pltpu.sync_copy(gamma_hbm, gam_v)
    pltpu.sync_copy(beta_hbm, bet_v)

    def chunk_body(g, _):
        base_tok = tok0 + g * C
        pltpu.sync_copy(idx_hbm.at[pl.ds(base_tok * 4, 4 * C)], idx_v)
        pltpu.async_copy(table_hbm.at[idx_v], rows_v, sem).wait()

        def token_body(t, _):
            def d_sum(d, carry):
                s, q = carry
                off = pl.ds(d * 16, 16)
                v = ((rows_v[4 * t + 0, off] + rows_v[4 * t + 1, off])
                     + (rows_v[4 * t + 2, off] + rows_v[4 * t + 3, off]))
                buf_v[t, off] = v
                return (s + v, q + v * v)

            zeros = jnp.zeros((16,), jnp.float32)
            s, q = lax.fori_loop(0, NV, d_sum, (zeros, zeros))
            lanes = lax.iota(jnp.int32, 16)
            for k in (8, 4, 2, 1):
                perm = lanes ^ k
                s = s + s.at[perm].get(mode="promise_in_bounds")
                q = q + q.at[perm].get(mode="promise_in_bounds")
            mvec = s * (1.0 / D)
            var = q * (1.0 / D) - mvec * mvec
            rstd = _rsqrt16(var + EPS)

            def d_norm(d, carry):
                off = pl.ds(d * 16, 16)
                v = buf_v[t, off]
                buf_v[t, off] = (v - mvec) * rstd * gam_v[off] + bet_v[off]
                return carry

            lax.fori_loop(0, NV, d_norm, 0)
            return 0

        lax.fori_loop(0, C, token_body, 0)
        pltpu.sync_copy(buf_v, out_hbm.at[pl.ds(base_tok, C)])
        return 0

    lax.fori_loop(0, NCHUNK, chunk_body, 0)


@jax.jit
def _sc_call(idx, table, gamma, beta):
    mesh = plsc.VectorSubcoreMesh(core_axis_name="c", subcore_axis_name="s")
    kfn = functools.partial(
        pl.kernel, mesh=mesh,
        out_type=jax.ShapeDtypeStruct((NTOK, D), jnp.float32),
        scratch_types=[
            pltpu.VMEM((4 * C,), jnp.int32),
            pltpu.VMEM((4 * C, D), jnp.float32),
            pltpu.VMEM((C, D), jnp.float32),
            pltpu.VMEM((D,), jnp.float32),
            pltpu.VMEM((D,), jnp.float32),
            pltpu.SemaphoreType.DMA,
        ],
    )(_sc_body)
    return kfn(idx, table, gamma, beta)


def kernel(bbox, x_emb, y_emb, gamma, beta):
    b, s, _ = bbox.shape
    offs = jnp.array([0, x_emb.shape[0], 0, x_emb.shape[0]], jnp.int32)
    idx = (bbox.reshape(b * s, 4) + offs).reshape(-1)
    table = jnp.concatenate([x_emb, y_emb], axis=0)
    out = _sc_call(idx, table, gamma, beta)
    return out.reshape(b, s, D)


# speedup vs baseline: 1.3234x; 1.3234x over previous
import functools

import jax
import jax.numpy as jnp
from jax import lax
from jax.experimental import pallas as pl
from jax.experimental.pallas import tpu as pltpu
from jax.experimental.pallas import tpu_sc as plsc

D = 1024
NTOK = 8192
NWORK = 32
TPW = NTOK // NWORK
C = 8
NCHUNK = TPW // C
NV = D // 16
UNROLL = 8
EPS = 1e-12


def _rsqrt16(v):
    i = lax.bitcast_convert_type(v, jnp.int32)
    i = jnp.int32(0x5F3759DF) - (i >> 1)
    y = lax.bitcast_convert_type(i, jnp.float32)
    for _ in range(3):
        y = y * (1.5 - 0.5 * v * y * y)
    return y


def _sc_body(idx_hbm, table_hbm, gamma_hbm, beta_hbm, out_hbm,
             idx_v, rows_v, buf_v, gam_v, bet_v, gsem0, gsem1):
    wid = lax.axis_index("s") * 2 + lax.axis_index("c")
    tok0 = wid * TPW

    pltpu.sync_copy(gamma_hbm, gam_v)
    pltpu.sync_copy(beta_hbm, bet_v)

    gsems = (gsem0, gsem1)

    def issue_gather(g, par):
        gc = jnp.minimum(g, NCHUNK - 1)
        base_tok = tok0 + gc * C
        pltpu.sync_copy(idx_hbm.at[pl.ds(base_tok * 4, 4 * C)], idx_v.at[par])
        return pltpu.async_copy(table_hbm.at[idx_v.at[par]], rows_v.at[par],
                                gsems[par])

    def compute_chunk(g, par):
        rows = rows_v.at[par]
        buf = buf_v.at[par]
        base_tok = tok0 + g * C

        def token_body(t, _):
            def d_sum(d8, carry):
                s, q = carry
                for j in range(UNROLL):
                    off = pl.ds(d8 * (16 * UNROLL) + j * 16, 16)
                    v = ((rows[4 * t + 0, off] + rows[4 * t + 1, off])
                         + (rows[4 * t + 2, off] + rows[4 * t + 3, off]))
                    buf[t, off] = v
                    s = s + v
                    q = q + v * v
                return (s, q)

            zeros = jnp.zeros((16,), jnp.float32)
            s, q = lax.fori_loop(0, NV // UNROLL, d_sum, (zeros, zeros))
            lanes = lax.iota(jnp.int32, 16)
            for k in (8, 4, 2, 1):
                perm = lanes ^ k
                s = s + s.at[perm].get(mode="promise_in_bounds")
                q = q + q.at[perm].get(mode="promise_in_bounds")
            mvec = s * (1.0 / D)
            var = q * (1.0 / D) - mvec * mvec
            rstd = _rsqrt16(var + EPS)

            def d_norm(d8, carry):
                for j in range(UNROLL):
                    off = pl.ds(d8 * (16 * UNROLL) + j * 16, 16)
                    v = buf[t, off]
                    buf[t, off] = (v - mvec) * rstd * gam_v[off] + bet_v[off]
                return carry

            lax.fori_loop(0, NV // UNROLL, d_norm, 0)
            return 0

        lax.fori_loop(0, C, token_body, 0)
        pltpu.sync_copy(buf, out_hbm.at[pl.ds(base_tok, C)])

    issue_gather(jnp.int32(0), 0).wait()
    h1 = issue_gather(jnp.int32(1), 1)

    def pipe_body(g2, _):
        compute_chunk(2 * g2, 0)
        h0n = issue_gather(2 * g2 + 2, 0)
        pltpu.make_async_copy(table_hbm.at[idx_v.at[1]], rows_v.at[1],
                              gsems[1]).wait()
        compute_chunk(2 * g2 + 1, 1)
        h1n = issue_gather(2 * g2 + 3, 1)
        pltpu.make_async_copy(table_hbm.at[idx_v.at[0]], rows_v.at[0],
                              gsems[0]).wait()
        return 0

    lax.fori_loop(0, NCHUNK // 2 - 1, pipe_body, 0)
    compute_chunk(NCHUNK - 2, 0)
    pltpu.make_async_copy(table_hbm.at[idx_v.at[1]], rows_v.at[1],
                          gsems[1]).wait()
    compute_chunk(NCHUNK - 1, 1)


@jax.jit
def _sc_call(idx, table, gamma, beta):
    mesh = plsc.VectorSubcoreMesh(core_axis_name="c", subcore_axis_name="s")
    kfn = functools.partial(
        pl.kernel, mesh=mesh,
        out_type=jax.ShapeDtypeStruct((NTOK, D), jnp.float32),
        scratch_types=[
            pltpu.VMEM((2, 4 * C), jnp.int32),
            pltpu.VMEM((2, 4 * C, D), jnp.float32),
            pltpu.VMEM((2, C, D), jnp.float32),
            pltpu.VMEM((D,), jnp.float32),
            pltpu.VMEM((D,), jnp.float32),
            pltpu.SemaphoreType.DMA,
            pltpu.SemaphoreType.DMA,
        ],
    )(_sc_body)
    return kfn(idx, table, gamma, beta)


def kernel(bbox, x_emb, y_emb, gamma, beta):
    b, s, _ = bbox.shape
    offs = jnp.array([0, x_emb.shape[0], 0, x_emb.shape[0]], jnp.int32)
    idx = (bbox.reshape(b * s, 4) + offs).reshape(-1)
    table = jnp.concatenate([x_emb, y_emb], axis=0)
    out = _sc_call(idx, table, gamma, beta)
    return out.reshape(b, s, D)


# speedup vs baseline: 3.4562x; 2.6116x over previous
import functools

import jax
import jax.numpy as jnp
from jax import lax
from jax.experimental import pallas as pl
from jax.experimental.pallas import tpu as pltpu
from jax.experimental.pallas import tpu_sc as plsc

D = 1024
NTOK = 8192
NWORK = 32
TPW = NTOK // NWORK
C = 8
NCHUNK = TPW // C
NV = D // 16
UNROLL = 8
EPS = 1e-12


def _rsqrt16(v):
    i = lax.bitcast_convert_type(v, jnp.int32)
    i = jnp.int32(0x5F3759DF) - (i >> 1)
    y = lax.bitcast_convert_type(i, jnp.float32)
    for _ in range(3):
        y = y * (1.5 - 0.5 * v * y * y)
    return y


def _lane_total(v):
    lanes = lax.iota(jnp.int32, 16)
    for k in (8, 4, 2, 1):
        v = v + v.at[lanes ^ k].get(mode="promise_in_bounds")
    return v


def _sc_body(idx_hbm, table_hbm, gamma_hbm, beta_hbm, out_hbm,
             idx_v, rows_v, buf_v, gam_v, bet_v,
             gsem0, gsem1, osem0, osem1):
    wid = lax.axis_index("s") * 2 + lax.axis_index("c")
    tok0 = wid * TPW

    pltpu.sync_copy(gamma_hbm, gam_v)
    pltpu.sync_copy(beta_hbm, bet_v)
    pltpu.sync_copy(idx_hbm.at[pl.ds(wid * NCHUNK, NCHUNK)], idx_v)

    def issue_gather(g, rows, gsem):
        return pltpu.async_copy(table_hbm.at[idx_v.at[g]], rows, gsem)

    def wait_gather(rows, gsem):
        pltpu.make_async_copy(table_hbm.at[idx_v.at[0]], rows, gsem).wait()

    def compute_chunk(g, rows, buf, osem):
        base_tok = tok0 + g * C

        def d_sum(d8, carry):
            s, q = carry
            ns, nq = [], []
            for t in range(C):
                st, qt = s[t], q[t]
                for j in range(UNROLL):
                    off = pl.ds(d8 * (16 * UNROLL) + j * 16, 16)
                    v = ((rows[4 * t + 0, off] + rows[4 * t + 1, off])
                         + (rows[4 * t + 2, off] + rows[4 * t + 3, off]))
                    buf[t, off] = v
                    st = st + v
                    qt = qt + v * v
                ns.append(st)
                nq.append(qt)
            return (tuple(ns), tuple(nq))

        zeros = jnp.zeros((16,), jnp.float32)
        s, q = lax.fori_loop(0, NV // UNROLL, d_sum,
                             ((zeros,) * C, (zeros,) * C))

        mvecs, rstds = [], []
        for t in range(C):
            mvec = _lane_total(s[t]) * (1.0 / D)
            var = _lane_total(q[t]) * (1.0 / D) - mvec * mvec
            mvecs.append(mvec)
            rstds.append(_rsqrt16(var + EPS))

        def d_norm(d8, carry):
            for j in range(UNROLL):
                off = pl.ds(d8 * (16 * UNROLL) + j * 16, 16)
                gv = gam_v[off]
                bv = bet_v[off]
                for t in range(C):
                    v = buf[t, off]
                    buf[t, off] = (v - mvecs[t]) * rstds[t] * gv + bv
            return carry

        lax.fori_loop(0, NV // UNROLL, d_norm, 0)
        return pltpu.async_copy(buf, out_hbm.at[pl.ds(base_tok, C)], osem)

    def wait_out(buf, osem):
        pltpu.make_async_copy(buf, out_hbm.at[pl.ds(tok0, C)], osem).wait()

    issue_gather(0, rows_v.at[0], gsem0)

    def pipe_body(g2, _):
        g = 2 * g2
        issue_gather(g + 1, rows_v.at[1], gsem1)
        wait_gather(rows_v.at[0], gsem0)

        @pl.when(g2 >= 1)
        def _():
            wait_out(buf_v.at[0], osem0)

        compute_chunk(g, rows_v.at[0], buf_v.at[0], osem0)

        @pl.when(g + 2 < NCHUNK)
        def _():
            issue_gather(g + 2, rows_v.at[0], gsem0)

        wait_gather(rows_v.at[1], gsem1)

        @pl.when(g2 >= 1)
        def _():
            wait_out(buf_v.at[1], osem1)

        compute_chunk(g + 1, rows_v.at[1], buf_v.at[1], osem1)
        return 0

    lax.fori_loop(0, NCHUNK // 2, pipe_body, 0)
    wait_out(buf_v.at[0], osem0)
    wait_out(buf_v.at[1], osem1)


@jax.jit
def _sc_call(idx, table, gamma, beta):
    mesh = plsc.VectorSubcoreMesh(core_axis_name="c", subcore_axis_name="s")
    kfn = functools.partial(
        pl.kernel, mesh=mesh,
        out_type=jax.ShapeDtypeStruct((NTOK, D), jnp.float32),
        scratch_types=[
            pltpu.VMEM((NCHUNK, 4 * C), jnp.int32),
            pltpu.VMEM((2, 4 * C, D), jnp.float32),
            pltpu.VMEM((2, C, D), jnp.float32),
            pltpu.VMEM((D,), jnp.float32),
            pltpu.VMEM((D,), jnp.float32),
            pltpu.SemaphoreType.DMA,
            pltpu.SemaphoreType.DMA,
            pltpu.SemaphoreType.DMA,
            pltpu.SemaphoreType.DMA,
        ],
    )(_sc_body)
    return kfn(idx, table, gamma, beta)


def kernel(bbox, x_emb, y_emb, gamma, beta):
    b, s, _ = bbox.shape
    offs = jnp.array([0, x_emb.shape[0], 0, x_emb.shape[0]], jnp.int32)
    idx = (bbox.reshape(b * s, 4) + offs).reshape(NWORK * NCHUNK, 4 * C)
    table = jnp.concatenate([x_emb, y_emb], axis=0)
    out = _sc_call(idx, table, gamma, beta)
    return out.reshape(b, s, D)
